# Initial kernel scaffold; baseline (speedup 1.0000x reference)
import jax
import jax.numpy as jnp
from jax.experimental import pallas as pl


def kernel(x, edge_index, W1, b1, W2, b2):
    raise NotImplementedError("write your pallas kernel here")


# speedup vs baseline: 5.0596x; 5.0596x over previous
import functools

import jax
import jax.numpy as jnp
from jax import lax
from jax.experimental import pallas as pl
from jax.experimental.pallas import tpu as pltpu
from jax.experimental.pallas import tpu_sc as plsc

N = 50000
E = 150000
R = 4
D = 128

NC = 2
NS = 16
NW = NC * NS
L = 16

ES = 4704
E_PAD = ES * NW
N_EPAD = E_PAD - E

CHUNK_SHIFT = 13
CHUNK = 1 << CHUNK_SHIFT
C = (N + CHUNK - 1) // CHUNK
AGG_ROWS = C * CHUNK
CH_PAD = CHUNK + 192
C8 = 8

BATCH = 128
CAP_B = (ES + BATCH - 1) // BATCH + 1
CAP = CAP_B * BATCH

SR = 3136
N_HPAD = NS * SR

MXU_BLK = 1024
NB = (N + MXU_BLK - 1) // MXU_BLK


def _mesh():
  return plsc.VectorSubcoreMesh(core_axis_name="c", subcore_axis_name="s")


def _prep_body(esrc, edst, bsrc, bdst, counts, degp,
               src_v, dst_v, hist_v, tmp_v, acc_v, bsrc_v, bdst_v,
               cnt_vv, stage):
  cid = lax.axis_index("c")
  sid = lax.axis_index("s")
  wid = sid * NC + cid
  base = wid * ES
  lanes = lax.iota(jnp.int32, L)
  ones = jnp.ones((L,), jnp.float32)
  zeros16 = jnp.zeros((L,), jnp.float32)

  def rel_body(r, carry):
    pltpu.sync_copy(esrc.at[pl.ds(r * E_PAD + base, ES)], src_v)
    pltpu.sync_copy(edst.at[pl.ds(r * E_PAD + base, ES)], dst_v)

    for kind in range(2):
      vec_ref = src_v if kind == 0 else dst_v

      def zh(i, _):
        hist_v[pl.ds(i * L, L)] = zeros16
        return 0
      lax.fori_loop(0, N_HPAD // L, zh, 0)

      def fill(i, _):
        idx = vec_ref[pl.ds(i * L, L)]
        valid = (base + i * L + lanes) < E
        val = jnp.where(valid, 1.0, 0.0).astype(jnp.float32)
        plsc.addupdate_scatter(hist_v, [idx], val)
        return 0
      lax.fori_loop(0, ES // L, fill, 0)

      pltpu.sync_copy(hist_v, stage.at[pl.ds(sid * N_HPAD, N_HPAD)])
      plsc.subcore_barrier()

      def za(i, _):
        acc_v[pl.ds(i * L, L)] = zeros16
        return 0
      lax.fori_loop(0, SR // L, za, 0)

      def red(j, _):
        pltpu.sync_copy(stage.at[pl.ds(j * N_HPAD + sid * SR, SR)], tmp_v)

        def addv(v, _):
          acc_v[pl.ds(v * L, L)] = acc_v[pl.ds(v * L, L)] + tmp_v[pl.ds(v * L, L)]
          return 0
        lax.fori_loop(0, SR // L, addv, 0)
        return 0
      lax.fori_loop(0, NS, red, 0)

      doff = ((cid * 2 + kind) * R + r) * N_HPAD + sid * SR
      pltpu.sync_copy(acc_v, degp.at[pl.ds(doff, SR)])
      plsc.subcore_barrier()

    def chunk_body(c, _):
      def compact(i, cnt):
        d = dst_v[pl.ds(i * L, L)]
        s = src_v[pl.ds(i * L, L)]
        m = lax.shift_right_logical(d, CHUNK_SHIFT) == c
        plsc.store_compressed(bsrc_v.at[pl.ds(cnt, L)], s + r * N, mask=m)
        plsc.store_compressed(bdst_v.at[pl.ds(cnt, L)], d - c * CHUNK, mask=m)
        return cnt + jnp.sum(m.astype(jnp.int32))
      cnt = lax.fori_loop(0, ES // L, compact, jnp.int32(0))

      for k in range(BATCH // L):
        pad_src = (wid * 61 + k * L + lanes) % jnp.int32(4096)
        pad_dst = CHUNK + ((wid * 7 + k * L + lanes) % jnp.int32(192))
        bsrc_v[pl.ds(cnt + k * L, L)] = pad_src + r * N
        bdst_v[pl.ds(cnt + k * L, L)] = pad_dst

      plsc.store_scatter(cnt_vv, [jnp.full((L,), r * C8 + c, jnp.int32)],
                         jnp.full((L,), 1, jnp.int32) * cnt,
                         mask=lanes == 0)
      boff = ((r * NW + wid) * C + c) * CAP
      pltpu.sync_copy(bsrc_v, bsrc.at[pl.ds(boff, CAP)])
      pltpu.sync_copy(bdst_v, bdst.at[pl.ds(boff, CAP)])
      return 0
    lax.fori_loop(0, C, chunk_body, 0)
    return carry

  lax.fori_loop(0, R, rel_body, 0)
  pltpu.sync_copy(cnt_vv, counts.at[pl.ds(wid * R * C8, R * C8)])


def _prep_call(edge_src, edge_dst):
  kfn = pl.kernel(
      _prep_body,
      out_type=[
          jax.ShapeDtypeStruct((R * NW * C * CAP,), jnp.int32),
          jax.ShapeDtypeStruct((R * NW * C * CAP,), jnp.int32),
          jax.ShapeDtypeStruct((NW * R * C8,), jnp.int32),
          jax.ShapeDtypeStruct((NC * 2 * R * N_HPAD,), jnp.float32),
      ],
      mesh=_mesh(),
      scratch_types=[
          pltpu.VMEM((ES,), jnp.int32),
          pltpu.VMEM((ES,), jnp.int32),
          pltpu.VMEM((N_HPAD,), jnp.float32),
          pltpu.VMEM((SR,), jnp.float32),
          pltpu.VMEM((SR,), jnp.float32),
          pltpu.VMEM((CAP,), jnp.int32),
          pltpu.VMEM((CAP,), jnp.int32),
          pltpu.VMEM((R * C8,), jnp.int32),
          pltpu.VMEM_SHARED((NS * N_HPAD,), jnp.float32),
      ],
      compiler_params=pltpu.CompilerParams(needs_layout_passes=False),
  )
  return kfn(edge_src, edge_dst)


def _agg_body(bsrc, bdst, counts, feat, agg,
              cnt_v, src_stage, dst_stage, src_b, dst_b, gbuf, zbuf, gsem, acc):
  cid = lax.axis_index("c")
  sid = lax.axis_index("s")
  zeros16 = jnp.zeros((L,), jnp.float32)

  def zz(i, _):
    zbuf[i // (D // L), pl.ds((i % (D // L)) * L, L)] = zeros16
    return 0
  lax.fori_loop(0, (BATCH * D) // L, zz, 0)

  pltpu.sync_copy(counts, cnt_v.at[pl.ds(0, NW * R * C8)])

  nch = (C - cid + 1) // NC

  def chunk_body(k, _):
    c = k * NC + cid

    def rel_body(r, _):
      for q in range(4):
        pltpu.sync_copy(zbuf, acc.at[pl.ds(sid * 512 + q * BATCH, BATCH)])
      plsc.subcore_barrier()

      for bi in range(2):
        t = sid * 2 + bi
        cnt = cnt_v[pl.ds(t * (R * C8) + r * C8 + c, L)][0]
        boff = ((r * NW + t) * C + c) * CAP
        pltpu.sync_copy(bsrc.at[pl.ds(boff, CAP)], src_stage)
        pltpu.sync_copy(bdst.at[pl.ds(boff, CAP)], dst_stage)
        nb = (cnt + (BATCH - 1)) // BATCH

        def batch_body(b, _):
          def cpi(j, _):
            src_b[pl.ds(j * L, L)] = src_stage[pl.ds(b * BATCH + j * L, L)]
            dst_b[pl.ds(j * L, L)] = dst_stage[pl.ds(b * BATCH + j * L, L)]
            return 0
          lax.fori_loop(0, BATCH // L, cpi, 0)
          pltpu.async_copy(feat.at[src_b], gbuf, gsem).wait()
          pltpu.sync_copy(gbuf, acc.at[dst_b], add=True)
          return 0
        lax.fori_loop(0, nb, batch_body, 0)

      plsc.subcore_barrier()
      pltpu.sync_copy(acc.at[pl.ds(sid * 512, 512)],
                      agg.at[r, pl.ds(c * CHUNK + sid * 512, 512)])
      return 0
    lax.fori_loop(0, R, rel_body, 0)
    return 0
  lax.fori_loop(0, nch, chunk_body, 0)


def _agg_call(bsrc5, bdst5, counts, feat_flat):
  kfn = pl.kernel(
      _agg_body,
      out_type=[
          jax.ShapeDtypeStruct((R, AGG_ROWS, D), jnp.float32),
      ],
      mesh=_mesh(),
      scratch_types=[
          pltpu.VMEM((NW * R * C8 + L,), jnp.int32),
          pltpu.VMEM((CAP,), jnp.int32),
          pltpu.VMEM((CAP,), jnp.int32),
          pltpu.VMEM((BATCH,), jnp.int32),
          pltpu.VMEM((BATCH,), jnp.int32),
          pltpu.VMEM((BATCH, D), jnp.float32),
          pltpu.VMEM((BATCH, D), jnp.float32),
          pltpu.SemaphoreType.DMA,
          pltpu.VMEM_SHARED((CH_PAD, D), jnp.float32),
      ],
      compiler_params=pltpu.CompilerParams(needs_layout_passes=False),
  )
  return kfn(bsrc5, bdst5, counts, feat_flat)


def _norm_body(degp_ref, out_ref):
  deg = degp_ref[0] + degp_ref[1]
  out_ref[...] = lax.rsqrt(jnp.maximum(deg, 1.0)).reshape(2 * R, N_HPAD)


def _norm_call(degp):
  return pl.pallas_call(
      _norm_body,
      out_shape=jax.ShapeDtypeStruct((2 * R, N_HPAD), jnp.float32),
  )(degp)


def _feat1_body(x_ref, n_ref, w_ref, out_ref):
  x = x_ref[...]
  for r in range(R):
    sn = n_ref[r]
    out_ref[r] = jnp.dot(x * sn[:, None], w_ref[r],
                         preferred_element_type=jnp.float32)


def _feat1_call(x, norms, W1):
  return pl.pallas_call(
      _feat1_body,
      grid=(NB,),
      in_specs=[
          pl.BlockSpec((MXU_BLK, D), lambda i: (i, 0)),
          pl.BlockSpec((2 * R, MXU_BLK), lambda i: (0, i)),
          pl.BlockSpec((R, D, D), lambda i: (0, 0, 0)),
      ],
      out_specs=pl.BlockSpec((R, MXU_BLK, D), lambda i: (0, i, 0)),
      out_shape=jax.ShapeDtypeStruct((R, N, D), jnp.float32),
  )(x, norms, W1)


def _mid_body(agg_ref, n_ref, b1_ref, w_ref, out_ref):
  bsum = jnp.sum(b1_ref[...], axis=0)
  h = jnp.broadcast_to(bsum[None, :], (MXU_BLK, D))
  for r in range(R):
    h = h + agg_ref[r] * n_ref[R + r][:, None]
  h = jnp.maximum(h, 0.0)
  for r in range(R):
    out_ref[r] = jnp.dot(h * n_ref[r][:, None], w_ref[r],
                         preferred_element_type=jnp.float32)


def _mid_call(agg1, norms, b1, W2):
  return pl.pallas_call(
      _mid_body,
      grid=(NB,),
      in_specs=[
          pl.BlockSpec((R, MXU_BLK, D), lambda i: (0, i, 0)),
          pl.BlockSpec((2 * R, MXU_BLK), lambda i: (0, i)),
          pl.BlockSpec((R, D), lambda i: (0, 0)),
          pl.BlockSpec((R, D, D), lambda i: (0, 0, 0)),
      ],
      out_specs=pl.BlockSpec((R, MXU_BLK, D), lambda i: (0, i, 0)),
      out_shape=jax.ShapeDtypeStruct((R, N, D), jnp.float32),
  )(agg1, norms, b1, W2)


def _fin_body(agg_ref, n_ref, b2_ref, out_ref):
  bsum = jnp.sum(b2_ref[...], axis=0)
  o = jnp.broadcast_to(bsum[None, :], (MXU_BLK, D))
  for r in range(R):
    o = o + agg_ref[r] * n_ref[R + r][:, None]
  out_ref[...] = o


def _fin_call(agg2, norms, b2):
  return pl.pallas_call(
      _fin_body,
      grid=(NB,),
      in_specs=[
          pl.BlockSpec((R, MXU_BLK, D), lambda i: (0, i, 0)),
          pl.BlockSpec((2 * R, MXU_BLK), lambda i: (0, i)),
          pl.BlockSpec((R, D), lambda i: (0, 0)),
      ],
      out_specs=pl.BlockSpec((MXU_BLK, D), lambda i: (i, 0)),
      out_shape=jax.ShapeDtypeStruct((N, D), jnp.float32),
  )(agg2, norms, b2)


@jax.jit
def kernel(x, edge_index, W1, b1, W2, b2):
  k = jnp.arange(N_EPAD, dtype=jnp.int32)
  pad_src = jnp.broadcast_to((k % 48)[None, :], (R, N_EPAD))
  pad_dst = jnp.broadcast_to((N + (k % 48))[None, :], (R, N_EPAD))
  edge_src = jnp.concatenate([edge_index[:, 0, :], pad_src], axis=1)
  edge_dst = jnp.concatenate([edge_index[:, 1, :], pad_dst], axis=1)

  bsrc, bdst, counts, degp = _prep_call(edge_src.reshape(R * E_PAD),
                                        edge_dst.reshape(R * E_PAD))
  norms = _norm_call(degp.reshape(NC, 2, R, N_HPAD))

  feat1 = _feat1_call(x, norms, W1).reshape(R * N, D)
  (agg1,) = _agg_call(bsrc, bdst, counts, feat1)
  feat2 = _mid_call(agg1, norms, b1, W2).reshape(R * N, D)
  (agg2,) = _agg_call(bsrc, bdst, counts, feat2)
  return _fin_call(agg2, norms, b2)


# speedup vs baseline: 5.6958x; 1.1257x over previous
import functools

import jax
import jax.numpy as jnp
from jax import lax
from jax.experimental import pallas as pl
from jax.experimental.pallas import tpu as pltpu
from jax.experimental.pallas import tpu_sc as plsc

N = 50000
E = 150000
R = 4
D = 128

NC = 2
NS = 16
NW = NC * NS
L = 16

ES = 4704
E_PAD = ES * NW
N_EPAD = E_PAD - E

CHUNK_SHIFT = 13
CHUNK = 1 << CHUNK_SHIFT
C = (N + CHUNK - 1) // CHUNK
AGG_ROWS = C * CHUNK
CH_PAD = CHUNK + 192
C8 = 8

BATCH = 128
CAP_B = (ES + BATCH - 1) // BATCH + 1
CAP = CAP_B * BATCH

SR = 3136
N_HPAD = NS * SR

MXU_BLK = 1024
NB = (N + MXU_BLK - 1) // MXU_BLK


def _mesh():
  return plsc.VectorSubcoreMesh(core_axis_name="c", subcore_axis_name="s")


def _prep_body(esrc, edst, bsrc, bdst, counts, degp,
               src_v, dst_v, hist_v, tmp_v, acc_v, bsrc_v, bdst_v,
               cnt_vv, stage):
  cid = lax.axis_index("c")
  sid = lax.axis_index("s")
  wid = sid * NC + cid
  base = wid * ES
  lanes = lax.iota(jnp.int32, L)
  ones = jnp.ones((L,), jnp.float32)
  zeros16 = jnp.zeros((L,), jnp.float32)

  def rel_body(r, carry):
    pltpu.sync_copy(esrc.at[pl.ds(r * E_PAD + base, ES)], src_v)
    pltpu.sync_copy(edst.at[pl.ds(r * E_PAD + base, ES)], dst_v)

    for kind in range(2):
      vec_ref = src_v if kind == 0 else dst_v

      def zh(i, _):
        hist_v[pl.ds(i * L, L)] = zeros16
        return 0
      lax.fori_loop(0, N_HPAD // L, zh, 0)

      def fill(i, _):
        idx = vec_ref[pl.ds(i * L, L)]
        valid = (base + i * L + lanes) < E
        val = jnp.where(valid, 1.0, 0.0).astype(jnp.float32)
        plsc.addupdate_scatter(hist_v, [idx], val)
        return 0
      lax.fori_loop(0, ES // L, fill, 0)

      pltpu.sync_copy(hist_v, stage.at[pl.ds(sid * N_HPAD, N_HPAD)])
      plsc.subcore_barrier()

      def za(i, _):
        acc_v[pl.ds(i * L, L)] = zeros16
        return 0
      lax.fori_loop(0, SR // L, za, 0)

      def red(j, _):
        pltpu.sync_copy(stage.at[pl.ds(j * N_HPAD + sid * SR, SR)], tmp_v)

        def addv(v, _):
          acc_v[pl.ds(v * L, L)] = acc_v[pl.ds(v * L, L)] + tmp_v[pl.ds(v * L, L)]
          return 0
        lax.fori_loop(0, SR // L, addv, 0)
        return 0
      lax.fori_loop(0, NS, red, 0)

      doff = ((cid * 2 + kind) * R + r) * N_HPAD + sid * SR
      pltpu.sync_copy(acc_v, degp.at[pl.ds(doff, SR)])
      plsc.subcore_barrier()

    def chunk_body(c, _):
      def compact(i, cnt):
        d = dst_v[pl.ds(i * L, L)]
        s = src_v[pl.ds(i * L, L)]
        m = lax.shift_right_logical(d, CHUNK_SHIFT) == c
        plsc.store_compressed(bsrc_v.at[pl.ds(cnt, L)], s + r * N, mask=m)
        plsc.store_compressed(bdst_v.at[pl.ds(cnt, L)], d - c * CHUNK, mask=m)
        return cnt + jnp.sum(m.astype(jnp.int32))
      cnt = lax.fori_loop(0, ES // L, compact, jnp.int32(0))

      for k in range(BATCH // L):
        pad_src = (wid * 61 + k * L + lanes) % jnp.int32(4096)
        pad_dst = CHUNK + ((wid * 7 + k * L + lanes) % jnp.int32(192))
        bsrc_v[pl.ds(cnt + k * L, L)] = pad_src + r * N
        bdst_v[pl.ds(cnt + k * L, L)] = pad_dst

      plsc.store_scatter(cnt_vv, [jnp.full((L,), r * C8 + c, jnp.int32)],
                         jnp.full((L,), 1, jnp.int32) * cnt,
                         mask=lanes == 0)
      boff = ((r * NW + wid) * C + c) * CAP
      pltpu.sync_copy(bsrc_v, bsrc.at[pl.ds(boff, CAP)])
      pltpu.sync_copy(bdst_v, bdst.at[pl.ds(boff, CAP)])
      return 0
    lax.fori_loop(0, C, chunk_body, 0)
    return carry

  lax.fori_loop(0, R, rel_body, 0)
  pltpu.sync_copy(cnt_vv, counts.at[pl.ds(wid * R * C8, R * C8)])


def _prep_call(edge_src, edge_dst):
  kfn = pl.kernel(
      _prep_body,
      out_type=[
          jax.ShapeDtypeStruct((R * NW * C * CAP,), jnp.int32),
          jax.ShapeDtypeStruct((R * NW * C * CAP,), jnp.int32),
          jax.ShapeDtypeStruct((NW * R * C8,), jnp.int32),
          jax.ShapeDtypeStruct((NC * 2 * R * N_HPAD,), jnp.float32),
      ],
      mesh=_mesh(),
      scratch_types=[
          pltpu.VMEM((ES,), jnp.int32),
          pltpu.VMEM((ES,), jnp.int32),
          pltpu.VMEM((N_HPAD,), jnp.float32),
          pltpu.VMEM((SR,), jnp.float32),
          pltpu.VMEM((SR,), jnp.float32),
          pltpu.VMEM((CAP,), jnp.int32),
          pltpu.VMEM((CAP,), jnp.int32),
          pltpu.VMEM((R * C8,), jnp.int32),
          pltpu.VMEM_SHARED((NS * N_HPAD,), jnp.float32),
      ],
      compiler_params=pltpu.CompilerParams(needs_layout_passes=False),
  )
  return kfn(edge_src, edge_dst)


def _agg_body(bsrc, bdst, counts, feat, agg,
              cnt_v, src_stage, dst_stage,
              src_b0, dst_b0, src_b1, dst_b1,
              gbuf0, gbuf1, zbuf, gsem0, gsem1, acc):
  cid = lax.axis_index("c")
  sid = lax.axis_index("s")
  zeros16 = jnp.zeros((L,), jnp.float32)

  def zz(i, _):
    zbuf[i // (D // L), pl.ds((i % (D // L)) * L, L)] = zeros16
    return 0
  lax.fori_loop(0, (BATCH * D) // L, zz, 0)

  pltpu.sync_copy(counts, cnt_v.at[pl.ds(0, NW * R * C8)])

  nch = 3 + cid

  def chunk_body(k, _):
    c = jnp.where(k < 3, k * NC + cid, 6)

    def rel_body(r, _):
      for q in range(4):
        pltpu.sync_copy(zbuf, acc.at[pl.ds(sid * 512 + q * BATCH, BATCH)])
      plsc.subcore_barrier()

      for bi in range(2):
        t = sid * 2 + bi
        cnt = cnt_v[pl.ds(t * (R * C8) + r * C8 + c, L)][0]
        boff = ((r * NW + t) * C + c) * CAP
        pltpu.sync_copy(bsrc.at[pl.ds(boff, CAP)], src_stage)
        pltpu.sync_copy(bdst.at[pl.ds(boff, CAP)], dst_stage)
        nb = (cnt + (BATCH - 1)) // BATCH

        def issue(b, src_b, dst_b, gbuf, gsem):
          def cpi(j, _):
            src_b[pl.ds(j * L, L)] = src_stage[pl.ds(b * BATCH + j * L, L)]
            dst_b[pl.ds(j * L, L)] = dst_stage[pl.ds(b * BATCH + j * L, L)]
            return 0
          lax.fori_loop(0, BATCH // L, cpi, 0)
          pltpu.async_copy(feat.at[src_b], gbuf, gsem)

        @pl.when(nb > 0)
        def _():
          issue(0, src_b0, dst_b0, gbuf0, gsem0)

        def batch_body(b, _):
          @pl.when(b % 2 == 0)
          def _():
            pltpu.make_async_copy(feat.at[src_b0], gbuf0, gsem0).wait()

            @pl.when(b + 1 < nb)
            def _():
              issue(b + 1, src_b1, dst_b1, gbuf1, gsem1)
            pltpu.sync_copy(gbuf0, acc.at[dst_b0], add=True)

          @pl.when(b % 2 == 1)
          def _():
            pltpu.make_async_copy(feat.at[src_b1], gbuf1, gsem1).wait()

            @pl.when(b + 1 < nb)
            def _():
              issue(b + 1, src_b0, dst_b0, gbuf0, gsem0)
            pltpu.sync_copy(gbuf1, acc.at[dst_b1], add=True)
          return 0
        lax.fori_loop(0, nb, batch_body, 0)

      plsc.subcore_barrier()
      pltpu.sync_copy(acc.at[pl.ds(sid * 512, 512)],
                      agg.at[r, pl.ds(c * CHUNK + sid * 512, 512)])
      return 0
    lax.fori_loop(0, R, rel_body, 0)
    return 0
  lax.fori_loop(0, nch, chunk_body, 0)


def _agg_call(bsrc5, bdst5, counts, feat_flat):
  kfn = pl.kernel(
      _agg_body,
      out_type=[
          jax.ShapeDtypeStruct((R, AGG_ROWS, D), jnp.float32),
      ],
      mesh=_mesh(),
      scratch_types=[
          pltpu.VMEM((NW * R * C8 + L,), jnp.int32),
          pltpu.VMEM((CAP,), jnp.int32),
          pltpu.VMEM((CAP,), jnp.int32),
          pltpu.VMEM((BATCH,), jnp.int32),
          pltpu.VMEM((BATCH,), jnp.int32),
          pltpu.VMEM((BATCH,), jnp.int32),
          pltpu.VMEM((BATCH,), jnp.int32),
          pltpu.VMEM((BATCH, D), jnp.float32),
          pltpu.VMEM((BATCH, D), jnp.float32),
          pltpu.VMEM((BATCH, D), jnp.float32),
          pltpu.SemaphoreType.DMA,
          pltpu.SemaphoreType.DMA,
          pltpu.VMEM_SHARED((CH_PAD, D), jnp.float32),
      ],
      compiler_params=pltpu.CompilerParams(needs_layout_passes=False),
  )
  return kfn(bsrc5, bdst5, counts, feat_flat)


def _norm_body(degp_ref, out_ref):
  deg = degp_ref[0] + degp_ref[1]
  out_ref[...] = lax.rsqrt(jnp.maximum(deg, 1.0)).reshape(2 * R, N_HPAD)


def _norm_call(degp):
  return pl.pallas_call(
      _norm_body,
      out_shape=jax.ShapeDtypeStruct((2 * R, N_HPAD), jnp.float32),
  )(degp)


def _feat1_body(x_ref, n_ref, w_ref, out_ref):
  x = x_ref[...]
  for r in range(R):
    sn = n_ref[r]
    out_ref[r] = jnp.dot(x * sn[:, None], w_ref[r],
                         preferred_element_type=jnp.float32)


def _feat1_call(x, norms, W1):
  return pl.pallas_call(
      _feat1_body,
      grid=(NB,),
      in_specs=[
          pl.BlockSpec((MXU_BLK, D), lambda i: (i, 0)),
          pl.BlockSpec((2 * R, MXU_BLK), lambda i: (0, i)),
          pl.BlockSpec((R, D, D), lambda i: (0, 0, 0)),
      ],
      out_specs=pl.BlockSpec((R, MXU_BLK, D), lambda i: (0, i, 0)),
      out_shape=jax.ShapeDtypeStruct((R, N, D), jnp.float32),
  )(x, norms, W1)


def _mid_body(agg_ref, n_ref, b1_ref, w_ref, out_ref):
  bsum = jnp.sum(b1_ref[...], axis=0)
  h = jnp.broadcast_to(bsum[None, :], (MXU_BLK, D))
  for r in range(R):
    h = h + agg_ref[r] * n_ref[R + r][:, None]
  h = jnp.maximum(h, 0.0)
  for r in range(R):
    out_ref[r] = jnp.dot(h * n_ref[r][:, None], w_ref[r],
                         preferred_element_type=jnp.float32)


def _mid_call(agg1, norms, b1, W2):
  return pl.pallas_call(
      _mid_body,
      grid=(NB,),
      in_specs=[
          pl.BlockSpec((R, MXU_BLK, D), lambda i: (0, i, 0)),
          pl.BlockSpec((2 * R, MXU_BLK), lambda i: (0, i)),
          pl.BlockSpec((R, D), lambda i: (0, 0)),
          pl.BlockSpec((R, D, D), lambda i: (0, 0, 0)),
      ],
      out_specs=pl.BlockSpec((R, MXU_BLK, D), lambda i: (0, i, 0)),
      out_shape=jax.ShapeDtypeStruct((R, N, D), jnp.float32),
  )(agg1, norms, b1, W2)


def _fin_body(agg_ref, n_ref, b2_ref, out_ref):
  bsum = jnp.sum(b2_ref[...], axis=0)
  o = jnp.broadcast_to(bsum[None, :], (MXU_BLK, D))
  for r in range(R):
    o = o + agg_ref[r] * n_ref[R + r][:, None]
  out_ref[...] = o


def _fin_call(agg2, norms, b2):
  return pl.pallas_call(
      _fin_body,
      grid=(NB,),
      in_specs=[
          pl.BlockSpec((R, MXU_BLK, D), lambda i: (0, i, 0)),
          pl.BlockSpec((2 * R, MXU_BLK), lambda i: (0, i)),
          pl.BlockSpec((R, D), lambda i: (0, 0)),
      ],
      out_specs=pl.BlockSpec((MXU_BLK, D), lambda i: (i, 0)),
      out_shape=jax.ShapeDtypeStruct((N, D), jnp.float32),
  )(agg2, norms, b2)


@jax.jit
def kernel(x, edge_index, W1, b1, W2, b2):
  k = jnp.arange(N_EPAD, dtype=jnp.int32)
  pad_src = jnp.broadcast_to((k % 48)[None, :], (R, N_EPAD))
  pad_dst = jnp.broadcast_to((N + (k % 48))[None, :], (R, N_EPAD))
  edge_src = jnp.concatenate([edge_index[:, 0, :], pad_src], axis=1)
  edge_dst = jnp.concatenate([edge_index[:, 1, :], pad_dst], axis=1)

  bsrc, bdst, counts, degp = _prep_call(edge_src.reshape(R * E_PAD),
                                        edge_dst.reshape(R * E_PAD))
  norms = _norm_call(degp.reshape(NC, 2, R, N_HPAD))

  feat1 = _feat1_call(x, norms, W1).reshape(R * N, D)
  (agg1,) = _agg_call(bsrc, bdst, counts, feat1)
  feat2 = _mid_call(agg1, norms, b1, W2).reshape(R * N, D)
  (agg2,) = _agg_call(bsrc, bdst, counts, feat2)
  return _fin_call(agg2, norms, b2)


# speedup vs baseline: 6.4935x; 1.1401x over previous
import functools

import jax
import jax.numpy as jnp
from jax import lax
from jax.experimental import pallas as pl
from jax.experimental.pallas import tpu as pltpu
from jax.experimental.pallas import tpu_sc as plsc

N = 50000
E = 150000
R = 4
D = 128

NC = 2
NS = 16
NW = NC * NS
L = 16

ES = 4704
E_PAD = ES * NW
N_EPAD = E_PAD - E

CHUNK_SHIFT = 13
CHUNK = 1 << CHUNK_SHIFT
C = (N + CHUNK - 1) // CHUNK
AGG_ROWS = C * CHUNK
CH_PAD = CHUNK + 192
C8 = 8

BATCH = 128
CAP_B = (ES + BATCH - 1) // BATCH + 1
CAP = CAP_B * BATCH

SR = 3136
N_HPAD = NS * SR

MXU_BLK = 1024
NB = (N + MXU_BLK - 1) // MXU_BLK


def _mesh():
  return plsc.VectorSubcoreMesh(core_axis_name="c", subcore_axis_name="s")


def _prep_body(esrc, edst, bsrc, bdst, counts, degp,
               src_v, dst_v, hist_v, tmp_v, tmp2_v, acc_v,
               bsrc_v, bdst_v, cnt_vv, tsem0, tsem1, stage):
  cid = lax.axis_index("c")
  sid = lax.axis_index("s")
  wid = sid * NC + cid
  base = wid * ES
  lanes = lax.iota(jnp.int32, L)
  ones = jnp.ones((L,), jnp.float32)
  zeros16 = jnp.zeros((L,), jnp.float32)

  def rel_body(r, carry):
    pltpu.sync_copy(esrc.at[pl.ds(r * E_PAD + base, ES)], src_v)
    pltpu.sync_copy(edst.at[pl.ds(r * E_PAD + base, ES)], dst_v)

    UN = 8
    for kind in range(2):
      vec_ref = src_v if kind == 0 else dst_v

      def zh(i, _):
        for u in range(UN):
          hist_v[pl.ds((i * UN + u) * L, L)] = zeros16
        return 0
      lax.fori_loop(0, N_HPAD // (L * UN), zh, 0)

      def fill(i, _):
        idx = vec_ref[pl.ds(i * L, L)]
        valid = (base + i * L + lanes) < E
        val = jnp.where(valid, 1.0, 0.0).astype(jnp.float32)
        plsc.addupdate_scatter(hist_v, [idx], val)
        return 0
      lax.fori_loop(0, ES // L, fill, 0)

      pltpu.sync_copy(hist_v, stage.at[pl.ds(sid * N_HPAD, N_HPAD)])
      plsc.subcore_barrier()

      def soff(j):
        return j * N_HPAD + sid * SR
      pltpu.sync_copy(stage.at[pl.ds(soff(0), SR)], acc_v)
      pltpu.async_copy(stage.at[pl.ds(soff(1), SR)], tmp_v, tsem0)
      for j in range(1, NS):
        cur, csem = (tmp_v, tsem0) if j % 2 == 1 else (tmp2_v, tsem1)
        pltpu.make_async_copy(stage.at[pl.ds(soff(j), SR)], cur, csem).wait()
        if j + 1 < NS:
          nxt, nsem = (tmp_v, tsem0) if j % 2 == 0 else (tmp2_v, tsem1)
          pltpu.async_copy(stage.at[pl.ds(soff(j + 1), SR)], nxt, nsem)

        def addv(v, _):
          for u in range(4):
            o = (v * 4 + u) * L
            acc_v[pl.ds(o, L)] = acc_v[pl.ds(o, L)] + cur[pl.ds(o, L)]
          return 0
        lax.fori_loop(0, SR // (L * 4), addv, 0)

      doff = ((cid * 2 + kind) * R + r) * N_HPAD + sid * SR
      pltpu.sync_copy(acc_v, degp.at[pl.ds(doff, SR)])
      plsc.subcore_barrier()

    def chunk_body(c, _):
      def compact(i, cnt):
        for u in range(2):
          d = dst_v[pl.ds((i * 2 + u) * L, L)]
          s = src_v[pl.ds((i * 2 + u) * L, L)]
          m = lax.shift_right_logical(d, CHUNK_SHIFT) == c
          plsc.store_compressed(bsrc_v.at[pl.ds(cnt, L)], s + r * N, mask=m)
          plsc.store_compressed(bdst_v.at[pl.ds(cnt, L)], d - c * CHUNK, mask=m)
          cnt = cnt + jnp.sum(m.astype(jnp.int32))
        return cnt
      cnt = lax.fori_loop(0, ES // (L * 2), compact, jnp.int32(0))

      for k in range(BATCH // L):
        pad_src = (wid * 61 + k * L + lanes) % jnp.int32(4096)
        pad_dst = CHUNK + ((wid * 7 + k * L + lanes) % jnp.int32(192))
        bsrc_v[pl.ds(cnt + k * L, L)] = pad_src + r * N
        bdst_v[pl.ds(cnt + k * L, L)] = pad_dst

      plsc.store_scatter(cnt_vv, [jnp.full((L,), r * C8 + c, jnp.int32)],
                         jnp.full((L,), 1, jnp.int32) * cnt,
                         mask=lanes == 0)
      boff = ((r * NW + wid) * C + c) * CAP
      pltpu.sync_copy(bsrc_v, bsrc.at[pl.ds(boff, CAP)])
      pltpu.sync_copy(bdst_v, bdst.at[pl.ds(boff, CAP)])
      return 0
    lax.fori_loop(0, C, chunk_body, 0)
    return carry

  lax.fori_loop(0, R, rel_body, 0)
  pltpu.sync_copy(cnt_vv, counts.at[pl.ds(wid * R * C8, R * C8)])


def _prep_call(edge_src, edge_dst):
  kfn = pl.kernel(
      _prep_body,
      out_type=[
          jax.ShapeDtypeStruct((R * NW * C * CAP,), jnp.int32),
          jax.ShapeDtypeStruct((R * NW * C * CAP,), jnp.int32),
          jax.ShapeDtypeStruct((NW * R * C8,), jnp.int32),
          jax.ShapeDtypeStruct((NC * 2 * R * N_HPAD,), jnp.float32),
      ],
      mesh=_mesh(),
      scratch_types=[
          pltpu.VMEM((ES,), jnp.int32),
          pltpu.VMEM((ES,), jnp.int32),
          pltpu.VMEM((N_HPAD,), jnp.float32),
          pltpu.VMEM((SR,), jnp.float32),
          pltpu.VMEM((SR,), jnp.float32),
          pltpu.VMEM((SR,), jnp.float32),
          pltpu.VMEM((CAP,), jnp.int32),
          pltpu.VMEM((CAP,), jnp.int32),
          pltpu.VMEM((R * C8,), jnp.int32),
          pltpu.SemaphoreType.DMA,
          pltpu.SemaphoreType.DMA,
          pltpu.VMEM_SHARED((NS * N_HPAD,), jnp.float32),
      ],
      compiler_params=pltpu.CompilerParams(needs_layout_passes=False),
  )
  return kfn(edge_src, edge_dst)


def _agg_body(bsrc, bdst, counts, feat, agg,
              cnt_v, src_stage, dst_stage,
              src_b0, dst_b0, src_b1, dst_b1,
              gbuf0, gbuf1, zbuf, gsem0, gsem1, acc):
  cid = lax.axis_index("c")
  sid = lax.axis_index("s")
  zeros16 = jnp.zeros((L,), jnp.float32)

  def zz(i, _):
    zbuf[i // (D // L), pl.ds((i % (D // L)) * L, L)] = zeros16
    return 0
  lax.fori_loop(0, (BATCH * D) // L, zz, 0)

  pltpu.sync_copy(counts, cnt_v.at[pl.ds(0, NW * R * C8)])

  nch = 3 + cid

  def chunk_body(k, _):
    c = jnp.where(k < 3, k * NC + cid, 6)

    def rel_body(r, _):
      for q in range(4):
        pltpu.sync_copy(zbuf, acc.at[pl.ds(sid * 512 + q * BATCH, BATCH)])
      plsc.subcore_barrier()

      for bi in range(2):
        t = sid * 2 + bi
        cnt = cnt_v[pl.ds(t * (R * C8) + r * C8 + c, L)][0]
        boff = ((r * NW + t) * C + c) * CAP
        pltpu.sync_copy(bsrc.at[pl.ds(boff, CAP)], src_stage)
        pltpu.sync_copy(bdst.at[pl.ds(boff, CAP)], dst_stage)
        nb = (cnt + (BATCH - 1)) // BATCH

        def issue(b, src_b, dst_b, gbuf, gsem):
          def cpi(j, _):
            src_b[pl.ds(j * L, L)] = src_stage[pl.ds(b * BATCH + j * L, L)]
            dst_b[pl.ds(j * L, L)] = dst_stage[pl.ds(b * BATCH + j * L, L)]
            return 0
          lax.fori_loop(0, BATCH // L, cpi, 0)
          pltpu.async_copy(feat.at[src_b], gbuf, gsem)

        @pl.when(nb > 0)
        def _():
          issue(0, src_b0, dst_b0, gbuf0, gsem0)

        def batch_body(b, _):
          @pl.when(b % 2 == 0)
          def _():
            pltpu.make_async_copy(feat.at[src_b0], gbuf0, gsem0).wait()

            @pl.when(b + 1 < nb)
            def _():
              issue(b + 1, src_b1, dst_b1, gbuf1, gsem1)
            pltpu.sync_copy(gbuf0, acc.at[dst_b0], add=True)

          @pl.when(b % 2 == 1)
          def _():
            pltpu.make_async_copy(feat.at[src_b1], gbuf1, gsem1).wait()

            @pl.when(b + 1 < nb)
            def _():
              issue(b + 1, src_b0, dst_b0, gbuf0, gsem0)
            pltpu.sync_copy(gbuf1, acc.at[dst_b1], add=True)
          return 0
        lax.fori_loop(0, nb, batch_body, 0)

      plsc.subcore_barrier()
      pltpu.sync_copy(acc.at[pl.ds(sid * 512, 512)],
                      agg.at[r, pl.ds(c * CHUNK + sid * 512, 512)])
      return 0
    lax.fori_loop(0, R, rel_body, 0)
    return 0
  lax.fori_loop(0, nch, chunk_body, 0)


def _agg_call(bsrc5, bdst5, counts, feat_flat):
  kfn = pl.kernel(
      _agg_body,
      out_type=[
          jax.ShapeDtypeStruct((R, AGG_ROWS, D), jnp.float32),
      ],
      mesh=_mesh(),
      scratch_types=[
          pltpu.VMEM((NW * R * C8 + L,), jnp.int32),
          pltpu.VMEM((CAP,), jnp.int32),
          pltpu.VMEM((CAP,), jnp.int32),
          pltpu.VMEM((BATCH,), jnp.int32),
          pltpu.VMEM((BATCH,), jnp.int32),
          pltpu.VMEM((BATCH,), jnp.int32),
          pltpu.VMEM((BATCH,), jnp.int32),
          pltpu.VMEM((BATCH, D), jnp.float32),
          pltpu.VMEM((BATCH, D), jnp.float32),
          pltpu.VMEM((BATCH, D), jnp.float32),
          pltpu.SemaphoreType.DMA,
          pltpu.SemaphoreType.DMA,
          pltpu.VMEM_SHARED((CH_PAD, D), jnp.float32),
      ],
      compiler_params=pltpu.CompilerParams(needs_layout_passes=False),
  )
  return kfn(bsrc5, bdst5, counts, feat_flat)


def _norm_body(degp_ref, out_ref):
  deg = degp_ref[0] + degp_ref[1]
  out_ref[...] = lax.rsqrt(jnp.maximum(deg, 1.0)).reshape(2 * R, N_HPAD)


def _norm_call(degp):
  return pl.pallas_call(
      _norm_body,
      out_shape=jax.ShapeDtypeStruct((2 * R, N_HPAD), jnp.float32),
  )(degp)


def _feat1_body(x_ref, n_ref, w_ref, out_ref):
  x = x_ref[...]
  for r in range(R):
    sn = n_ref[r]
    out_ref[r] = jnp.dot(x * sn[:, None], w_ref[r],
                         preferred_element_type=jnp.float32)


def _feat1_call(x, norms, W1):
  return pl.pallas_call(
      _feat1_body,
      grid=(NB,),
      in_specs=[
          pl.BlockSpec((MXU_BLK, D), lambda i: (i, 0)),
          pl.BlockSpec((2 * R, MXU_BLK), lambda i: (0, i)),
          pl.BlockSpec((R, D, D), lambda i: (0, 0, 0)),
      ],
      out_specs=pl.BlockSpec((R, MXU_BLK, D), lambda i: (0, i, 0)),
      out_shape=jax.ShapeDtypeStruct((R, N, D), jnp.float32),
  )(x, norms, W1)


def _mid_body(agg_ref, n_ref, b1_ref, w_ref, out_ref):
  bsum = jnp.sum(b1_ref[...], axis=0)
  h = jnp.broadcast_to(bsum[None, :], (MXU_BLK, D))
  for r in range(R):
    h = h + agg_ref[r] * n_ref[R + r][:, None]
  h = jnp.maximum(h, 0.0)
  for r in range(R):
    out_ref[r] = jnp.dot(h * n_ref[r][:, None], w_ref[r],
                         preferred_element_type=jnp.float32)


def _mid_call(agg1, norms, b1, W2):
  return pl.pallas_call(
      _mid_body,
      grid=(NB,),
      in_specs=[
          pl.BlockSpec((R, MXU_BLK, D), lambda i: (0, i, 0)),
          pl.BlockSpec((2 * R, MXU_BLK), lambda i: (0, i)),
          pl.BlockSpec((R, D), lambda i: (0, 0)),
          pl.BlockSpec((R, D, D), lambda i: (0, 0, 0)),
      ],
      out_specs=pl.BlockSpec((R, MXU_BLK, D), lambda i: (0, i, 0)),
      out_shape=jax.ShapeDtypeStruct((R, N, D), jnp.float32),
  )(agg1, norms, b1, W2)


def _fin_body(agg_ref, n_ref, b2_ref, out_ref):
  bsum = jnp.sum(b2_ref[...], axis=0)
  o = jnp.broadcast_to(bsum[None, :], (MXU_BLK, D))
  for r in range(R):
    o = o + agg_ref[r] * n_ref[R + r][:, None]
  out_ref[...] = o


def _fin_call(agg2, norms, b2):
  return pl.pallas_call(
      _fin_body,
      grid=(NB,),
      in_specs=[
          pl.BlockSpec((R, MXU_BLK, D), lambda i: (0, i, 0)),
          pl.BlockSpec((2 * R, MXU_BLK), lambda i: (0, i)),
          pl.BlockSpec((R, D), lambda i: (0, 0)),
      ],
      out_specs=pl.BlockSpec((MXU_BLK, D), lambda i: (i, 0)),
      out_shape=jax.ShapeDtypeStruct((N, D), jnp.float32),
  )(agg2, norms, b2)


@jax.jit
def kernel(x, edge_index, W1, b1, W2, b2):
  k = jnp.arange(N_EPAD, dtype=jnp.int32)
  pad_src = jnp.broadcast_to((k % 48)[None, :], (R, N_EPAD))
  pad_dst = jnp.broadcast_to((N + (k % 48))[None, :], (R, N_EPAD))
  edge_src = jnp.concatenate([edge_index[:, 0, :], pad_src], axis=1)
  edge_dst = jnp.concatenate([edge_index[:, 1, :], pad_dst], axis=1)

  bsrc, bdst, counts, degp = _prep_call(edge_src.reshape(R * E_PAD),
                                        edge_dst.reshape(R * E_PAD))
  norms = _norm_call(degp.reshape(NC, 2, R, N_HPAD))

  feat1 = _feat1_call(x, norms, W1).reshape(R * N, D)
  (agg1,) = _agg_call(bsrc, bdst, counts, feat1)
  feat2 = _mid_call(agg1, norms, b1, W2).reshape(R * N, D)
  (agg2,) = _agg_call(bsrc, bdst, counts, feat2)
  return _fin_call(agg2, norms, b2)


# speedup vs baseline: 7.0132x; 1.0800x over previous
import functools

import jax
import jax.numpy as jnp
from jax import lax
from jax.experimental import pallas as pl
from jax.experimental.pallas import tpu as pltpu
from jax.experimental.pallas import tpu_sc as plsc

N = 50000
E = 150000
R = 4
D = 128

NC = 2
NS = 16
NW = NC * NS
L = 16

ES = 4704
E_PAD = ES * NW
N_EPAD = E_PAD - E

CHUNK_SHIFT = 13
CHUNK = 1 << CHUNK_SHIFT
C = (N + CHUNK - 1) // CHUNK
AGG_ROWS = C * CHUNK
CH_PAD = CHUNK + 192
C8 = 8

BATCH = 128
CAP_B = (ES + BATCH - 1) // BATCH + 1
CAP = CAP_B * BATCH

SR = 3136
N_HPAD = NS * SR

MXU_BLK = 1024
NB = (N + MXU_BLK - 1) // MXU_BLK


def _mesh():
  return plsc.VectorSubcoreMesh(core_axis_name="c", subcore_axis_name="s")


def _hist_body(edges, degp,
               src_v, dst_v, hist_v, tmp_v, tmp2_v, acc_v,
               tsem0, tsem1, stage):
  cid = lax.axis_index("c")
  sid = lax.axis_index("s")
  wid = sid * NC + cid
  base = jnp.minimum(wid * ES, E - ES)
  skip = wid * ES - base
  lanes = lax.iota(jnp.int32, L)
  zeros16 = jnp.zeros((L,), jnp.float32)

  def rel_body(r, carry):
    pltpu.sync_copy(edges.at[pl.ds(2 * r * E + base, ES)], src_v)
    pltpu.sync_copy(edges.at[pl.ds((2 * r + 1) * E + base, ES)], dst_v)

    UN = 8
    for kind in range(2):
      vec_ref = src_v if kind == 0 else dst_v

      def zh(i, _):
        for u in range(UN):
          hist_v[pl.ds((i * UN + u) * L, L)] = zeros16
        return 0
      lax.fori_loop(0, N_HPAD // (L * UN), zh, 0)

      def fill(i, _):
        idx = vec_ref[pl.ds(i * L, L)]
        valid = (i * L + lanes) >= skip
        val = jnp.where(valid, 1.0, 0.0).astype(jnp.float32)
        plsc.addupdate_scatter(hist_v, [idx], val)
        return 0
      lax.fori_loop(0, ES // L, fill, 0)

      pltpu.sync_copy(hist_v, stage.at[pl.ds(sid * N_HPAD, N_HPAD)])
      plsc.subcore_barrier()

      def soff(j):
        return j * N_HPAD + sid * SR
      pltpu.sync_copy(stage.at[pl.ds(soff(0), SR)], acc_v)
      pltpu.async_copy(stage.at[pl.ds(soff(1), SR)], tmp_v, tsem0)
      for j in range(1, NS):
        cur, csem = (tmp_v, tsem0) if j % 2 == 1 else (tmp2_v, tsem1)
        pltpu.make_async_copy(stage.at[pl.ds(soff(j), SR)], cur, csem).wait()
        if j + 1 < NS:
          nxt, nsem = (tmp_v, tsem0) if j % 2 == 0 else (tmp2_v, tsem1)
          pltpu.async_copy(stage.at[pl.ds(soff(j + 1), SR)], nxt, nsem)

        def addv(v, _):
          for u in range(4):
            o = (v * 4 + u) * L
            acc_v[pl.ds(o, L)] = acc_v[pl.ds(o, L)] + cur[pl.ds(o, L)]
          return 0
        lax.fori_loop(0, SR // (L * 4), addv, 0)

      doff = ((cid * 2 + kind) * R + r) * N_HPAD + sid * SR
      pltpu.sync_copy(acc_v, degp.at[pl.ds(doff, SR)])
      plsc.subcore_barrier()
    return carry

  lax.fori_loop(0, R, rel_body, 0)


def _hist_call(edge_flat):
  kfn = pl.kernel(
      _hist_body,
      out_type=[
          jax.ShapeDtypeStruct((NC * 2 * R * N_HPAD,), jnp.float32),
      ],
      mesh=_mesh(),
      scratch_types=[
          pltpu.VMEM((ES,), jnp.int32),
          pltpu.VMEM((ES,), jnp.int32),
          pltpu.VMEM((N_HPAD,), jnp.float32),
          pltpu.VMEM((SR,), jnp.float32),
          pltpu.VMEM((SR,), jnp.float32),
          pltpu.VMEM((SR,), jnp.float32),
          pltpu.SemaphoreType.DMA,
          pltpu.SemaphoreType.DMA,
          pltpu.VMEM_SHARED((NS * N_HPAD,), jnp.float32),
      ],
      compiler_params=pltpu.CompilerParams(needs_layout_passes=False),
  )
  return kfn(edge_flat)


def _bucket_body(edges, bsrc, bdst, counts,
                 src_v, dst_v, bsrc_v, bdst_v, cnt_vv):
  cid = lax.axis_index("c")
  sid = lax.axis_index("s")
  wid = sid * NC + cid
  base = jnp.minimum(wid * ES, E - ES)
  skip = wid * ES - base
  lanes = lax.iota(jnp.int32, L)

  def rel_body(r, carry):
    pltpu.sync_copy(edges.at[pl.ds(2 * r * E + base, ES)], src_v)
    pltpu.sync_copy(edges.at[pl.ds((2 * r + 1) * E + base, ES)], dst_v)

    def chunk_body(c, _):
      def compact(i, cnt):
        for u in range(2):
          d = dst_v[pl.ds((i * 2 + u) * L, L)]
          s = src_v[pl.ds((i * 2 + u) * L, L)]
          m = (lax.shift_right_logical(d, CHUNK_SHIFT) == c) & (
              ((i * 2 + u) * L + lanes) >= skip)
          plsc.store_compressed(bsrc_v.at[pl.ds(cnt, L)], s + r * N, mask=m)
          plsc.store_compressed(bdst_v.at[pl.ds(cnt, L)], d - c * CHUNK, mask=m)
          cnt = cnt + jnp.sum(m.astype(jnp.int32))
        return cnt
      cnt = lax.fori_loop(0, ES // (L * 2), compact, jnp.int32(0))

      for k in range(BATCH // L):
        pad_src = (wid * 61 + k * L + lanes) % jnp.int32(4096)
        pad_dst = CHUNK + ((wid * 7 + k * L + lanes) % jnp.int32(192))
        bsrc_v[pl.ds(cnt + k * L, L)] = pad_src + r * N
        bdst_v[pl.ds(cnt + k * L, L)] = pad_dst

      plsc.store_scatter(cnt_vv, [jnp.full((L,), r * C8 + c, jnp.int32)],
                         jnp.full((L,), 1, jnp.int32) * cnt,
                         mask=lanes == 0)
      boff = ((r * NW + wid) * C + c) * CAP
      pltpu.sync_copy(bsrc_v, bsrc.at[pl.ds(boff, CAP)])
      pltpu.sync_copy(bdst_v, bdst.at[pl.ds(boff, CAP)])
      return 0
    lax.fori_loop(0, C, chunk_body, 0)
    return carry

  lax.fori_loop(0, R, rel_body, 0)
  pltpu.sync_copy(cnt_vv, counts.at[pl.ds(wid * R * C8, R * C8)])


def _bucket_call(edge_flat):
  kfn = pl.kernel(
      _bucket_body,
      out_type=[
          jax.ShapeDtypeStruct((R * NW * C * CAP,), jnp.int32),
          jax.ShapeDtypeStruct((R * NW * C * CAP,), jnp.int32),
          jax.ShapeDtypeStruct((NW * R * C8,), jnp.int32),
      ],
      mesh=_mesh(),
      scratch_types=[
          pltpu.VMEM((ES,), jnp.int32),
          pltpu.VMEM((ES,), jnp.int32),
          pltpu.VMEM((CAP,), jnp.int32),
          pltpu.VMEM((CAP,), jnp.int32),
          pltpu.VMEM((R * C8,), jnp.int32),
      ],
      compiler_params=pltpu.CompilerParams(needs_layout_passes=False),
  )
  return kfn(edge_flat)


def _agg_body(bsrc, bdst, counts, feat, agg,
              cnt_v, src_stage, dst_stage,
              src_b0, dst_b0, src_b1, dst_b1,
              gbuf0, gbuf1, zbuf, gsem0, gsem1, acc):
  cid = lax.axis_index("c")
  sid = lax.axis_index("s")
  zeros16 = jnp.zeros((L,), jnp.float32)

  def zz(i, _):
    zbuf[i // (D // L), pl.ds((i % (D // L)) * L, L)] = zeros16
    return 0
  lax.fori_loop(0, (BATCH * D) // L, zz, 0)

  pltpu.sync_copy(counts, cnt_v.at[pl.ds(0, NW * R * C8)])

  nch = 3 + cid

  def chunk_body(k, _):
    c = jnp.where(k < 3, k * NC + cid, 6)

    def rel_body(r, _):
      for q in range(4):
        pltpu.sync_copy(zbuf, acc.at[pl.ds(sid * 512 + q * BATCH, BATCH)])
      plsc.subcore_barrier()

      for bi in range(2):
        t = sid * 2 + bi
        cnt = cnt_v[pl.ds(t * (R * C8) + r * C8 + c, L)][0]
        boff = ((r * NW + t) * C + c) * CAP
        pltpu.sync_copy(bsrc.at[pl.ds(boff, CAP)], src_stage)
        pltpu.sync_copy(bdst.at[pl.ds(boff, CAP)], dst_stage)
        nb = (cnt + (BATCH - 1)) // BATCH

        def issue(b, src_b, dst_b, gbuf, gsem):
          def cpi(j, _):
            src_b[pl.ds(j * L, L)] = src_stage[pl.ds(b * BATCH + j * L, L)]
            dst_b[pl.ds(j * L, L)] = dst_stage[pl.ds(b * BATCH + j * L, L)]
            return 0
          lax.fori_loop(0, BATCH // L, cpi, 0)
          pltpu.async_copy(feat.at[src_b], gbuf, gsem)

        @pl.when(nb > 0)
        def _():
          issue(0, src_b0, dst_b0, gbuf0, gsem0)

        def batch_body(b, _):
          @pl.when(b % 2 == 0)
          def _():
            pltpu.make_async_copy(feat.at[src_b0], gbuf0, gsem0).wait()

            @pl.when(b + 1 < nb)
            def _():
              issue(b + 1, src_b1, dst_b1, gbuf1, gsem1)
            pltpu.sync_copy(gbuf0, acc.at[dst_b0], add=True)

          @pl.when(b % 2 == 1)
          def _():
            pltpu.make_async_copy(feat.at[src_b1], gbuf1, gsem1).wait()

            @pl.when(b + 1 < nb)
            def _():
              issue(b + 1, src_b0, dst_b0, gbuf0, gsem0)
            pltpu.sync_copy(gbuf1, acc.at[dst_b1], add=True)
          return 0
        lax.fori_loop(0, nb, batch_body, 0)

      plsc.subcore_barrier()
      pltpu.sync_copy(acc.at[pl.ds(sid * 512, 512)],
                      agg.at[r, pl.ds(c * CHUNK + sid * 512, 512)])
      return 0
    lax.fori_loop(0, R, rel_body, 0)
    return 0
  lax.fori_loop(0, nch, chunk_body, 0)


def _agg_call(bsrc5, bdst5, counts, feat_flat):
  kfn = pl.kernel(
      _agg_body,
      out_type=[
          jax.ShapeDtypeStruct((R, AGG_ROWS, D), jnp.float32),
      ],
      mesh=_mesh(),
      scratch_types=[
          pltpu.VMEM((NW * R * C8 + L,), jnp.int32),
          pltpu.VMEM((CAP,), jnp.int32),
          pltpu.VMEM((CAP,), jnp.int32),
          pltpu.VMEM((BATCH,), jnp.int32),
          pltpu.VMEM((BATCH,), jnp.int32),
          pltpu.VMEM((BATCH,), jnp.int32),
          pltpu.VMEM((BATCH,), jnp.int32),
          pltpu.VMEM((BATCH, D), jnp.float32),
          pltpu.VMEM((BATCH, D), jnp.float32),
          pltpu.VMEM((BATCH, D), jnp.float32),
          pltpu.SemaphoreType.DMA,
          pltpu.SemaphoreType.DMA,
          pltpu.VMEM_SHARED((CH_PAD, D), jnp.float32),
      ],
      compiler_params=pltpu.CompilerParams(needs_layout_passes=False),
  )
  return kfn(bsrc5, bdst5, counts, feat_flat)


def _srcnorm(d_ref, r):
  return lax.rsqrt(jnp.maximum(d_ref[r] + d_ref[2 * R + r], 1.0))


def _dstnorm(d_ref, r):
  return lax.rsqrt(jnp.maximum(d_ref[R + r] + d_ref[3 * R + r], 1.0))


def _feat1_body(x_ref, d_ref, w_ref, out_ref):
  x = x_ref[...]
  for r in range(R):
    sn = _srcnorm(d_ref, r)
    out_ref[r] = jnp.dot(x * sn[:, None], w_ref[r],
                         preferred_element_type=jnp.float32)


def _feat1_call(x, deg2, W1):
  return pl.pallas_call(
      _feat1_body,
      grid=(NB,),
      in_specs=[
          pl.BlockSpec((MXU_BLK, D), lambda i: (i, 0)),
          pl.BlockSpec((NC * 2 * R, MXU_BLK), lambda i: (0, i)),
          pl.BlockSpec((R, D, D), lambda i: (0, 0, 0)),
      ],
      out_specs=pl.BlockSpec((R, MXU_BLK, D), lambda i: (0, i, 0)),
      out_shape=jax.ShapeDtypeStruct((R, N, D), jnp.float32),
  )(x, deg2, W1)


def _mid_body(agg_ref, d_ref, b1_ref, w_ref, out_ref):
  bsum = jnp.sum(b1_ref[...], axis=0)
  h = jnp.broadcast_to(bsum[None, :], (MXU_BLK, D))
  for r in range(R):
    h = h + agg_ref[r] * _dstnorm(d_ref, r)[:, None]
  h = jnp.maximum(h, 0.0)
  for r in range(R):
    out_ref[r] = jnp.dot(h * _srcnorm(d_ref, r)[:, None], w_ref[r],
                         preferred_element_type=jnp.float32)


def _mid_call(agg1, deg2, b1, W2):
  return pl.pallas_call(
      _mid_body,
      grid=(NB,),
      in_specs=[
          pl.BlockSpec((R, MXU_BLK, D), lambda i: (0, i, 0)),
          pl.BlockSpec((NC * 2 * R, MXU_BLK), lambda i: (0, i)),
          pl.BlockSpec((R, D), lambda i: (0, 0)),
          pl.BlockSpec((R, D, D), lambda i: (0, 0, 0)),
      ],
      out_specs=pl.BlockSpec((R, MXU_BLK, D), lambda i: (0, i, 0)),
      out_shape=jax.ShapeDtypeStruct((R, N, D), jnp.float32),
  )(agg1, deg2, b1, W2)


def _fin_body(agg_ref, d_ref, b2_ref, out_ref):
  bsum = jnp.sum(b2_ref[...], axis=0)
  o = jnp.broadcast_to(bsum[None, :], (MXU_BLK, D))
  for r in range(R):
    o = o + agg_ref[r] * _dstnorm(d_ref, r)[:, None]
  out_ref[...] = o


def _fin_call(agg2, deg2, b2):
  return pl.pallas_call(
      _fin_body,
      grid=(NB,),
      in_specs=[
          pl.BlockSpec((R, MXU_BLK, D), lambda i: (0, i, 0)),
          pl.BlockSpec((NC * 2 * R, MXU_BLK), lambda i: (0, i)),
          pl.BlockSpec((R, D), lambda i: (0, 0)),
      ],
      out_specs=pl.BlockSpec((MXU_BLK, D), lambda i: (i, 0)),
      out_shape=jax.ShapeDtypeStruct((N, D), jnp.float32),
  )(agg2, deg2, b2)


@jax.jit
def kernel(x, edge_index, W1, b1, W2, b2):
  edge_flat = edge_index.reshape(R * 2 * E)
  (degp,) = _hist_call(edge_flat)
  bsrc, bdst, counts = _bucket_call(edge_flat)
  deg2 = degp.reshape(NC * 2 * R, N_HPAD)

  feat1 = _feat1_call(x, deg2, W1).reshape(R * N, D)
  (agg1,) = _agg_call(bsrc, bdst, counts, feat1)
  feat2 = _mid_call(agg1, deg2, b1, W2).reshape(R * N, D)
  (agg2,) = _agg_call(bsrc, bdst, counts, feat2)
  return _fin_call(agg2, deg2, b2)


# speedup vs baseline: 7.0210x; 1.0011x over previous
import functools

import jax
import jax.numpy as jnp
from jax import lax
from jax.experimental import pallas as pl
from jax.experimental.pallas import tpu as pltpu
from jax.experimental.pallas import tpu_sc as plsc

N = 50000
E = 150000
R = 4
D = 128

NC = 2
NS = 16
NW = NC * NS
L = 16

ES = 4704
E_PAD = ES * NW
N_EPAD = E_PAD - E

CHUNK_SHIFT = 13
CHUNK = 1 << CHUNK_SHIFT
C = (N + CHUNK - 1) // CHUNK
AGG_ROWS = C * CHUNK
CH_PAD = CHUNK + 192
C8 = 8

BATCH = 128
CAP_B = (ES + BATCH - 1) // BATCH + 1
CAP = CAP_B * BATCH

SR = 3136
N_HPAD = NS * SR

MXU_BLK = 1024
NB = (N + MXU_BLK - 1) // MXU_BLK


def _mesh():
  return plsc.VectorSubcoreMesh(core_axis_name="c", subcore_axis_name="s")


def _hist_body(edges, degp,
               src_v, dst_v, hist_v, tmp_v, tmp2_v, acc_v,
               tsem0, tsem1, stage):
  cid = lax.axis_index("c")
  sid = lax.axis_index("s")
  wid = sid * NC + cid
  base = jnp.minimum(wid * ES, E - ES)
  skip = wid * ES - base
  lanes = lax.iota(jnp.int32, L)
  zeros16 = jnp.zeros((L,), jnp.float32)

  def rel_body(r, carry):
    pltpu.sync_copy(edges.at[pl.ds(2 * r * E + base, ES)], src_v)
    pltpu.sync_copy(edges.at[pl.ds((2 * r + 1) * E + base, ES)], dst_v)

    UN = 8
    for kind in range(2):
      vec_ref = src_v if kind == 0 else dst_v

      def zh(i, _):
        for u in range(UN):
          hist_v[pl.ds((i * UN + u) * L, L)] = zeros16
        return 0
      lax.fori_loop(0, N_HPAD // (L * UN), zh, 0)

      def fill(i, _):
        idx = vec_ref[pl.ds(i * L, L)]
        valid = (i * L + lanes) >= skip
        val = jnp.where(valid, 1.0, 0.0).astype(jnp.float32)
        plsc.addupdate_scatter(hist_v, [idx], val)
        return 0
      lax.fori_loop(0, ES // L, fill, 0)

      pltpu.sync_copy(hist_v, stage.at[pl.ds(sid * N_HPAD, N_HPAD)])
      plsc.subcore_barrier()

      def soff(j):
        return j * N_HPAD + sid * SR
      pltpu.sync_copy(stage.at[pl.ds(soff(0), SR)], acc_v)
      pltpu.async_copy(stage.at[pl.ds(soff(1), SR)], tmp_v, tsem0)
      for j in range(1, NS):
        cur, csem = (tmp_v, tsem0) if j % 2 == 1 else (tmp2_v, tsem1)
        pltpu.make_async_copy(stage.at[pl.ds(soff(j), SR)], cur, csem).wait()
        if j + 1 < NS:
          nxt, nsem = (tmp_v, tsem0) if j % 2 == 0 else (tmp2_v, tsem1)
          pltpu.async_copy(stage.at[pl.ds(soff(j + 1), SR)], nxt, nsem)

        def addv(v, _):
          for u in range(4):
            o = (v * 4 + u) * L
            acc_v[pl.ds(o, L)] = acc_v[pl.ds(o, L)] + cur[pl.ds(o, L)]
          return 0
        lax.fori_loop(0, SR // (L * 4), addv, 0)

      doff = ((cid * 2 + kind) * R + r) * N_HPAD + sid * SR
      pltpu.sync_copy(acc_v, degp.at[pl.ds(doff, SR)])
      plsc.subcore_barrier()
    return carry

  lax.fori_loop(0, R, rel_body, 0)


def _hist_call(edge_flat):
  kfn = pl.kernel(
      _hist_body,
      out_type=[
          jax.ShapeDtypeStruct((NC * 2 * R * N_HPAD,), jnp.float32),
      ],
      mesh=_mesh(),
      scratch_types=[
          pltpu.VMEM((ES,), jnp.int32),
          pltpu.VMEM((ES,), jnp.int32),
          pltpu.VMEM((N_HPAD,), jnp.float32),
          pltpu.VMEM((SR,), jnp.float32),
          pltpu.VMEM((SR,), jnp.float32),
          pltpu.VMEM((SR,), jnp.float32),
          pltpu.SemaphoreType.DMA,
          pltpu.SemaphoreType.DMA,
          pltpu.VMEM_SHARED((NS * N_HPAD,), jnp.float32),
      ],
      compiler_params=pltpu.CompilerParams(needs_layout_passes=False),
  )
  return kfn(edge_flat)


def _bucket_body(edges, bsrc, bdst, counts,
                 src_v, dst_v, bsrc_v, bdst_v, cnt_vv):
  cid = lax.axis_index("c")
  sid = lax.axis_index("s")
  wid = sid * NC + cid
  base = jnp.minimum(wid * ES, E - ES)
  skip = wid * ES - base
  lanes = lax.iota(jnp.int32, L)

  def rel_body(r, carry):
    pltpu.sync_copy(edges.at[pl.ds(2 * r * E + base, ES)], src_v)
    pltpu.sync_copy(edges.at[pl.ds((2 * r + 1) * E + base, ES)], dst_v)

    def chunk_body(c, _):
      def compact(i, cnt):
        for u in range(2):
          d = dst_v[pl.ds((i * 2 + u) * L, L)]
          s = src_v[pl.ds((i * 2 + u) * L, L)]
          m = (lax.shift_right_logical(d, CHUNK_SHIFT) == c) & (
              ((i * 2 + u) * L + lanes) >= skip)
          plsc.store_compressed(bsrc_v.at[pl.ds(cnt, L)], s + r * N, mask=m)
          plsc.store_compressed(bdst_v.at[pl.ds(cnt, L)], d - c * CHUNK, mask=m)
          cnt = cnt + jnp.sum(m.astype(jnp.int32))
        return cnt
      cnt = lax.fori_loop(0, ES // (L * 2), compact, jnp.int32(0))

      for k in range(BATCH // L):
        pad_src = (wid * 61 + k * L + lanes) % jnp.int32(4096)
        pad_dst = CHUNK + ((wid * 7 + k * L + lanes) % jnp.int32(192))
        bsrc_v[pl.ds(cnt + k * L, L)] = pad_src + r * N
        bdst_v[pl.ds(cnt + k * L, L)] = pad_dst

      plsc.store_scatter(cnt_vv, [jnp.full((L,), r * C8 + c, jnp.int32)],
                         jnp.full((L,), 1, jnp.int32) * cnt,
                         mask=lanes == 0)
      boff = ((r * NW + wid) * C + c) * CAP
      pltpu.sync_copy(bsrc_v, bsrc.at[pl.ds(boff, CAP)])
      pltpu.sync_copy(bdst_v, bdst.at[pl.ds(boff, CAP)])
      return 0
    lax.fori_loop(0, C, chunk_body, 0)
    return carry

  lax.fori_loop(0, R, rel_body, 0)
  pltpu.sync_copy(cnt_vv, counts.at[pl.ds(wid * R * C8, R * C8)])


def _bucket_call(edge_flat):
  kfn = pl.kernel(
      _bucket_body,
      out_type=[
          jax.ShapeDtypeStruct((R * NW * C * CAP,), jnp.int32),
          jax.ShapeDtypeStruct((R * NW * C * CAP,), jnp.int32),
          jax.ShapeDtypeStruct((NW * R * C8,), jnp.int32),
      ],
      mesh=_mesh(),
      scratch_types=[
          pltpu.VMEM((ES,), jnp.int32),
          pltpu.VMEM((ES,), jnp.int32),
          pltpu.VMEM((CAP,), jnp.int32),
          pltpu.VMEM((CAP,), jnp.int32),
          pltpu.VMEM((R * C8,), jnp.int32),
      ],
      compiler_params=pltpu.CompilerParams(needs_layout_passes=False),
  )
  return kfn(edge_flat)


def _agg_body(bsrc, bdst, counts, feat, agg,
              cnt_v, src_stage, dst_stage,
              src_b0, dst_b0, src_b1, dst_b1,
              gbuf0, gbuf1, zbuf, gsem0, gsem1, ssem0, ssem1, acc):
  cid = lax.axis_index("c")
  sid = lax.axis_index("s")
  zeros16 = jnp.zeros((L,), jnp.float32)

  def zz(i, _):
    zbuf[i // (D // L), pl.ds((i % (D // L)) * L, L)] = zeros16
    return 0
  lax.fori_loop(0, (BATCH * D) // L, zz, 0)

  pltpu.sync_copy(counts, cnt_v.at[pl.ds(0, NW * R * C8)])

  nch = 3 + cid

  def chunk_body(k, _):
    c = jnp.where(k < 3, k * NC + cid, 6)

    def rel_body(r, _):
      for q in range(4):
        pltpu.sync_copy(zbuf, acc.at[pl.ds(sid * 512 + q * BATCH, BATCH)])
      plsc.subcore_barrier()

      for bi in range(2):
        t = sid * 2 + bi
        cnt = cnt_v[pl.ds(t * (R * C8) + r * C8 + c, L)][0]
        boff = ((r * NW + t) * C + c) * CAP
        pltpu.sync_copy(bsrc.at[pl.ds(boff, CAP)], src_stage)
        pltpu.sync_copy(bdst.at[pl.ds(boff, CAP)], dst_stage)
        nb = (cnt + (BATCH - 1)) // BATCH

        bufs = ((src_b0, dst_b0, gbuf0, gsem0, ssem0),
                (src_b1, dst_b1, gbuf1, gsem1, ssem1))

        def issue(b, src_b, dst_b, gbuf, gsem):
          def cpi(j, _):
            src_b[pl.ds(j * L, L)] = src_stage[pl.ds(b * BATCH + j * L, L)]
            dst_b[pl.ds(j * L, L)] = dst_stage[pl.ds(b * BATCH + j * L, L)]
            return 0
          lax.fori_loop(0, BATCH // L, cpi, 0)
          pltpu.async_copy(feat.at[src_b], gbuf, gsem)

        @pl.when(nb > 0)
        def _():
          issue(0, src_b0, dst_b0, gbuf0, gsem0)

        def batch_body(b, _):
          for par in range(2):
            sbP, dbP, gbP, gsP, ssP = bufs[par]
            sbQ, dbQ, gbQ, gsQ, ssQ = bufs[1 - par]

            @pl.when(b % 2 == par)
            def _():
              pltpu.make_async_copy(feat.at[sbP], gbP, gsP).wait()
              pltpu.async_copy(gbP, acc.at[dbP], ssP, add=True)

              @pl.when(b >= 1)
              def _():
                pltpu.make_async_copy(gbQ, acc.at[dbQ], ssQ).wait()

              @pl.when(b + 1 < nb)
              def _():
                issue(b + 1, sbQ, dbQ, gbQ, gsQ)
          return 0
        lax.fori_loop(0, nb, batch_body, 0)

        @pl.when(nb > 0)
        def _():
          for par in range(2):
            sbP, dbP, gbP, gsP, ssP = bufs[par]

            @pl.when((nb - 1) % 2 == par)
            def _():
              pltpu.make_async_copy(gbP, acc.at[dbP], ssP).wait()

      plsc.subcore_barrier()
      pltpu.sync_copy(acc.at[pl.ds(sid * 512, 512)],
                      agg.at[r, pl.ds(c * CHUNK + sid * 512, 512)])
      return 0
    lax.fori_loop(0, R, rel_body, 0)
    return 0
  lax.fori_loop(0, nch, chunk_body, 0)


def _agg_call(bsrc5, bdst5, counts, feat_flat):
  kfn = pl.kernel(
      _agg_body,
      out_type=[
          jax.ShapeDtypeStruct((R, AGG_ROWS, D), jnp.float32),
      ],
      mesh=_mesh(),
      scratch_types=[
          pltpu.VMEM((NW * R * C8 + L,), jnp.int32),
          pltpu.VMEM((CAP,), jnp.int32),
          pltpu.VMEM((CAP,), jnp.int32),
          pltpu.VMEM((BATCH,), jnp.int32),
          pltpu.VMEM((BATCH,), jnp.int32),
          pltpu.VMEM((BATCH,), jnp.int32),
          pltpu.VMEM((BATCH,), jnp.int32),
          pltpu.VMEM((BATCH, D), jnp.float32),
          pltpu.VMEM((BATCH, D), jnp.float32),
          pltpu.VMEM((BATCH, D), jnp.float32),
          pltpu.SemaphoreType.DMA,
          pltpu.SemaphoreType.DMA,
          pltpu.SemaphoreType.DMA,
          pltpu.SemaphoreType.DMA,
          pltpu.VMEM_SHARED((CH_PAD, D), jnp.float32),
      ],
      compiler_params=pltpu.CompilerParams(needs_layout_passes=False),
  )
  return kfn(bsrc5, bdst5, counts, feat_flat)


def _srcnorm(d_ref, r):
  return lax.rsqrt(jnp.maximum(d_ref[r] + d_ref[2 * R + r], 1.0))


def _dstnorm(d_ref, r):
  return lax.rsqrt(jnp.maximum(d_ref[R + r] + d_ref[3 * R + r], 1.0))


def _feat1_body(x_ref, d_ref, w_ref, out_ref):
  x = x_ref[...]
  for r in range(R):
    sn = _srcnorm(d_ref, r)
    out_ref[r] = jnp.dot(x * sn[:, None], w_ref[r],
                         preferred_element_type=jnp.float32)


def _feat1_call(x, deg2, W1):
  return pl.pallas_call(
      _feat1_body,
      grid=(NB,),
      in_specs=[
          pl.BlockSpec((MXU_BLK, D), lambda i: (i, 0)),
          pl.BlockSpec((NC * 2 * R, MXU_BLK), lambda i: (0, i)),
          pl.BlockSpec((R, D, D), lambda i: (0, 0, 0)),
      ],
      out_specs=pl.BlockSpec((R, MXU_BLK, D), lambda i: (0, i, 0)),
      out_shape=jax.ShapeDtypeStruct((R, N, D), jnp.float32),
  )(x, deg2, W1)


def _mid_body(agg_ref, d_ref, b1_ref, w_ref, out_ref):
  bsum = jnp.sum(b1_ref[...], axis=0)
  h = jnp.broadcast_to(bsum[None, :], (MXU_BLK, D))
  for r in range(R):
    h = h + agg_ref[r] * _dstnorm(d_ref, r)[:, None]
  h = jnp.maximum(h, 0.0)
  for r in range(R):
    out_ref[r] = jnp.dot(h * _srcnorm(d_ref, r)[:, None], w_ref[r],
                         preferred_element_type=jnp.float32)


def _mid_call(agg1, deg2, b1, W2):
  return pl.pallas_call(
      _mid_body,
      grid=(NB,),
      in_specs=[
          pl.BlockSpec((R, MXU_BLK, D), lambda i: (0, i, 0)),
          pl.BlockSpec((NC * 2 * R, MXU_BLK), lambda i: (0, i)),
          pl.BlockSpec((R, D), lambda i: (0, 0)),
          pl.BlockSpec((R, D, D), lambda i: (0, 0, 0)),
      ],
      out_specs=pl.BlockSpec((R, MXU_BLK, D), lambda i: (0, i, 0)),
      out_shape=jax.ShapeDtypeStruct((R, N, D), jnp.float32),
  )(agg1, deg2, b1, W2)


def _fin_body(agg_ref, d_ref, b2_ref, out_ref):
  bsum = jnp.sum(b2_ref[...], axis=0)
  o = jnp.broadcast_to(bsum[None, :], (MXU_BLK, D))
  for r in range(R):
    o = o + agg_ref[r] * _dstnorm(d_ref, r)[:, None]
  out_ref[...] = o


def _fin_call(agg2, deg2, b2):
  return pl.pallas_call(
      _fin_body,
      grid=(NB,),
      in_specs=[
          pl.BlockSpec((R, MXU_BLK, D), lambda i: (0, i, 0)),
          pl.BlockSpec((NC * 2 * R, MXU_BLK), lambda i: (0, i)),
          pl.BlockSpec((R, D), lambda i: (0, 0)),
      ],
      out_specs=pl.BlockSpec((MXU_BLK, D), lambda i: (i, 0)),
      out_shape=jax.ShapeDtypeStruct((N, D), jnp.float32),
  )(agg2, deg2, b2)


@jax.jit
def kernel(x, edge_index, W1, b1, W2, b2):
  edge_flat = edge_index.reshape(R * 2 * E)
  (degp,) = _hist_call(edge_flat)
  bsrc, bdst, counts = _bucket_call(edge_flat)
  deg2 = degp.reshape(NC * 2 * R, N_HPAD)

  feat1 = _feat1_call(x, deg2, W1).reshape(R * N, D)
  (agg1,) = _agg_call(bsrc, bdst, counts, feat1)
  feat2 = _mid_call(agg1, deg2, b1, W2).reshape(R * N, D)
  (agg2,) = _agg_call(bsrc, bdst, counts, feat2)
  return _fin_call(agg2, deg2, b2)


# speedup vs baseline: 7.2232x; 1.0288x over previous
import functools

import jax
import jax.numpy as jnp
from jax import lax
from jax.experimental import pallas as pl
from jax.experimental.pallas import tpu as pltpu
from jax.experimental.pallas import tpu_sc as plsc

N = 50000
E = 150000
R = 4
D = 128

NC = 2
NS = 16
NW = NC * NS
L = 16

ES = 4704
E_PAD = ES * NW
N_EPAD = E_PAD - E

CHUNK_SHIFT = 13
CHUNK = 1 << CHUNK_SHIFT
C = (N + CHUNK - 1) // CHUNK
AGG_ROWS = C * CHUNK
CH_PAD = CHUNK + 192
C8 = 8

BATCH = 128
CAP_B = (ES + BATCH - 1) // BATCH + 1
CAP = CAP_B * BATCH

SR = 3136
N_HPAD = NS * SR

MXU_BLK = 1024
NB = (N + MXU_BLK - 1) // MXU_BLK


def _mesh():
  return plsc.VectorSubcoreMesh(core_axis_name="c", subcore_axis_name="s")


def _hist_body(edges, degp,
               src_v, dst_v, hist_v, tmp_v, tmp2_v, acc_v,
               tsem0, tsem1, stage):
  cid = lax.axis_index("c")
  sid = lax.axis_index("s")
  wid = sid * NC + cid
  base = jnp.minimum(wid * ES, E - ES)
  skip = wid * ES - base
  lanes = lax.iota(jnp.int32, L)
  zeros16 = jnp.zeros((L,), jnp.float32)

  def rel_body(r, carry):
    pltpu.sync_copy(edges.at[pl.ds(2 * r * E + base, ES)], src_v)
    pltpu.sync_copy(edges.at[pl.ds((2 * r + 1) * E + base, ES)], dst_v)

    UN = 8
    for kind in range(2):
      vec_ref = src_v if kind == 0 else dst_v

      def zh(i, _):
        for u in range(UN):
          hist_v[pl.ds((i * UN + u) * L, L)] = zeros16
        return 0
      lax.fori_loop(0, N_HPAD // (L * UN), zh, 0)

      def fill(i, _):
        idx = vec_ref[pl.ds(i * L, L)]
        valid = (i * L + lanes) >= skip
        val = jnp.where(valid, 1.0, 0.0).astype(jnp.float32)
        plsc.addupdate_scatter(hist_v, [idx], val)
        return 0
      lax.fori_loop(0, ES // L, fill, 0)

      pltpu.sync_copy(hist_v, stage.at[pl.ds(sid * N_HPAD, N_HPAD)])
      plsc.subcore_barrier()

      def soff(j):
        return j * N_HPAD + sid * SR
      pltpu.sync_copy(stage.at[pl.ds(soff(0), SR)], acc_v)
      pltpu.async_copy(stage.at[pl.ds(soff(1), SR)], tmp_v, tsem0)
      for j in range(1, NS):
        cur, csem = (tmp_v, tsem0) if j % 2 == 1 else (tmp2_v, tsem1)
        pltpu.make_async_copy(stage.at[pl.ds(soff(j), SR)], cur, csem).wait()
        if j + 1 < NS:
          nxt, nsem = (tmp_v, tsem0) if j % 2 == 0 else (tmp2_v, tsem1)
          pltpu.async_copy(stage.at[pl.ds(soff(j + 1), SR)], nxt, nsem)

        def addv(v, _):
          for u in range(4):
            o = (v * 4 + u) * L
            acc_v[pl.ds(o, L)] = acc_v[pl.ds(o, L)] + cur[pl.ds(o, L)]
          return 0
        lax.fori_loop(0, SR // (L * 4), addv, 0)

      doff = ((cid * 2 + kind) * R + r) * N_HPAD + sid * SR
      pltpu.sync_copy(acc_v, degp.at[pl.ds(doff, SR)])
      plsc.subcore_barrier()
    return carry

  lax.fori_loop(0, R, rel_body, 0)


def _hist_call(edge_flat):
  kfn = pl.kernel(
      _hist_body,
      out_type=[
          jax.ShapeDtypeStruct((NC * 2 * R * N_HPAD,), jnp.float32),
      ],
      mesh=_mesh(),
      scratch_types=[
          pltpu.VMEM((ES,), jnp.int32),
          pltpu.VMEM((ES,), jnp.int32),
          pltpu.VMEM((N_HPAD,), jnp.float32),
          pltpu.VMEM((SR,), jnp.float32),
          pltpu.VMEM((SR,), jnp.float32),
          pltpu.VMEM((SR,), jnp.float32),
          pltpu.SemaphoreType.DMA,
          pltpu.SemaphoreType.DMA,
          pltpu.VMEM_SHARED((NS * N_HPAD,), jnp.float32),
      ],
      compiler_params=pltpu.CompilerParams(needs_layout_passes=False),
  )
  return kfn(edge_flat)


def _bucket_body(edges, bsrc, bdst, counts,
                 src_v, dst_v, bsrc_v, bdst_v, cnt_vv):
  cid = lax.axis_index("c")
  sid = lax.axis_index("s")
  wid = sid * NC + cid
  base = jnp.minimum(wid * ES, E - ES)
  skip = wid * ES - base
  lanes = lax.iota(jnp.int32, L)

  def rel_body(r, carry):
    pltpu.sync_copy(edges.at[pl.ds(2 * r * E + base, ES)], src_v)
    pltpu.sync_copy(edges.at[pl.ds((2 * r + 1) * E + base, ES)], dst_v)

    def chunk_body(c, _):
      def compact(i, cnt):
        for u in range(2):
          d = dst_v[pl.ds((i * 2 + u) * L, L)]
          s = src_v[pl.ds((i * 2 + u) * L, L)]
          m = (lax.shift_right_logical(d, CHUNK_SHIFT) == c) & (
              ((i * 2 + u) * L + lanes) >= skip)
          plsc.store_compressed(bsrc_v.at[pl.ds(cnt, L)], s + r * N, mask=m)
          plsc.store_compressed(bdst_v.at[pl.ds(cnt, L)], d - c * CHUNK, mask=m)
          cnt = cnt + jnp.sum(m.astype(jnp.int32))
        return cnt
      cnt = lax.fori_loop(0, ES // (L * 2), compact, jnp.int32(0))

      for k in range(BATCH // L):
        pad_src = (wid * 61 + k * L + lanes) % jnp.int32(4096)
        pad_dst = CHUNK + ((wid * 7 + k * L + lanes) % jnp.int32(192))
        bsrc_v[pl.ds(cnt + k * L, L)] = pad_src + r * N
        bdst_v[pl.ds(cnt + k * L, L)] = pad_dst

      plsc.store_scatter(cnt_vv, [jnp.full((L,), r * C8 + c, jnp.int32)],
                         jnp.full((L,), 1, jnp.int32) * cnt,
                         mask=lanes == 0)
      boff = ((r * NW + wid) * C + c) * CAP
      pltpu.sync_copy(bsrc_v, bsrc.at[pl.ds(boff, CAP)])
      pltpu.sync_copy(bdst_v, bdst.at[pl.ds(boff, CAP)])
      return 0
    lax.fori_loop(0, C, chunk_body, 0)
    return carry

  lax.fori_loop(0, R, rel_body, 0)
  pltpu.sync_copy(cnt_vv, counts.at[pl.ds(wid * R * C8, R * C8)])


def _bucket_call(edge_flat):
  kfn = pl.kernel(
      _bucket_body,
      out_type=[
          jax.ShapeDtypeStruct((R * NW * C * CAP,), jnp.int32),
          jax.ShapeDtypeStruct((R * NW * C * CAP,), jnp.int32),
          jax.ShapeDtypeStruct((NW * R * C8,), jnp.int32),
      ],
      mesh=_mesh(),
      scratch_types=[
          pltpu.VMEM((ES,), jnp.int32),
          pltpu.VMEM((ES,), jnp.int32),
          pltpu.VMEM((CAP,), jnp.int32),
          pltpu.VMEM((CAP,), jnp.int32),
          pltpu.VMEM((R * C8,), jnp.int32),
      ],
      compiler_params=pltpu.CompilerParams(needs_layout_passes=False),
  )
  return kfn(edge_flat)


def _agg_body(bsrc, bdst, counts, feat, agg,
              cnt_v, src_stage, dst_stage,
              src_b0, dst_b0, src_b1, dst_b1,
              gbuf0, gbuf1, zbuf, gsem0, gsem1, ssem0, ssem1, acc):
  cid = lax.axis_index("c")
  sid = lax.axis_index("s")
  zeros16 = jnp.zeros((L,), jnp.float32)

  def zz(i, _):
    zbuf[i // (D // L), pl.ds((i % (D // L)) * L, L)] = zeros16
    return 0
  lax.fori_loop(0, (BATCH * D) // L, zz, 0)

  pltpu.sync_copy(counts, cnt_v.at[pl.ds(0, NW * R * C8)])

  nch = 3 + cid

  def chunk_body(k, _):
    c = jnp.where(k < 3, k * NC + cid, 6)

    live = sid * 512 < N - c * CHUNK

    def rel_body(r, _):
      @pl.when(live)
      def _():
        for q in range(4):
          pltpu.sync_copy(zbuf, acc.at[pl.ds(sid * 512 + q * BATCH, BATCH)])
      plsc.subcore_barrier()

      for bi in range(2):
        t = sid * 2 + bi
        cnt = cnt_v[pl.ds(t * (R * C8) + r * C8 + c, L)][0]
        boff = ((r * NW + t) * C + c) * CAP
        pltpu.sync_copy(bsrc.at[pl.ds(boff, CAP)], src_stage)
        pltpu.sync_copy(bdst.at[pl.ds(boff, CAP)], dst_stage)
        nb = (cnt + (BATCH - 1)) // BATCH

        bufs = ((src_b0, dst_b0, gbuf0, gsem0, ssem0),
                (src_b1, dst_b1, gbuf1, gsem1, ssem1))

        def issue(b, src_b, dst_b, gbuf, gsem):
          def cpi(j, _):
            src_b[pl.ds(j * L, L)] = src_stage[pl.ds(b * BATCH + j * L, L)]
            dst_b[pl.ds(j * L, L)] = dst_stage[pl.ds(b * BATCH + j * L, L)]
            return 0
          lax.fori_loop(0, BATCH // L, cpi, 0)
          pltpu.async_copy(feat.at[src_b], gbuf, gsem)

        @pl.when(nb > 0)
        def _():
          issue(0, src_b0, dst_b0, gbuf0, gsem0)

        def batch_body(b, _):
          for par in range(2):
            sbP, dbP, gbP, gsP, ssP = bufs[par]
            sbQ, dbQ, gbQ, gsQ, ssQ = bufs[1 - par]

            @pl.when(b % 2 == par)
            def _():
              pltpu.make_async_copy(feat.at[sbP], gbP, gsP).wait()
              pltpu.async_copy(gbP, acc.at[dbP], ssP, add=True)

              @pl.when(b >= 1)
              def _():
                pltpu.make_async_copy(gbQ, acc.at[dbQ], ssQ).wait()

              @pl.when(b + 1 < nb)
              def _():
                issue(b + 1, sbQ, dbQ, gbQ, gsQ)
          return 0
        lax.fori_loop(0, nb, batch_body, 0)

        @pl.when(nb > 0)
        def _():
          for par in range(2):
            sbP, dbP, gbP, gsP, ssP = bufs[par]

            @pl.when((nb - 1) % 2 == par)
            def _():
              pltpu.make_async_copy(gbP, acc.at[dbP], ssP).wait()

      plsc.subcore_barrier()

      @pl.when(live)
      def _():
        pltpu.sync_copy(acc.at[pl.ds(sid * 512, 512)],
                        agg.at[r, pl.ds(c * CHUNK + sid * 512, 512)])
      return 0
    lax.fori_loop(0, R, rel_body, 0)
    return 0
  lax.fori_loop(0, nch, chunk_body, 0)


def _agg_call(bsrc5, bdst5, counts, feat_flat):
  kfn = pl.kernel(
      _agg_body,
      out_type=[
          jax.ShapeDtypeStruct((R, AGG_ROWS, D), jnp.float32),
      ],
      mesh=_mesh(),
      scratch_types=[
          pltpu.VMEM((NW * R * C8 + L,), jnp.int32),
          pltpu.VMEM((CAP,), jnp.int32),
          pltpu.VMEM((CAP,), jnp.int32),
          pltpu.VMEM((BATCH,), jnp.int32),
          pltpu.VMEM((BATCH,), jnp.int32),
          pltpu.VMEM((BATCH,), jnp.int32),
          pltpu.VMEM((BATCH,), jnp.int32),
          pltpu.VMEM((BATCH, D), jnp.float32),
          pltpu.VMEM((BATCH, D), jnp.float32),
          pltpu.VMEM((BATCH, D), jnp.float32),
          pltpu.SemaphoreType.DMA,
          pltpu.SemaphoreType.DMA,
          pltpu.SemaphoreType.DMA,
          pltpu.SemaphoreType.DMA,
          pltpu.VMEM_SHARED((CH_PAD, D), jnp.float32),
      ],
      compiler_params=pltpu.CompilerParams(needs_layout_passes=False),
  )
  return kfn(bsrc5, bdst5, counts, feat_flat)


def _srcnorm(d_ref, r):
  return lax.rsqrt(jnp.maximum(d_ref[r] + d_ref[2 * R + r], 1.0))


def _dstnorm(d_ref, r):
  return lax.rsqrt(jnp.maximum(d_ref[R + r] + d_ref[3 * R + r], 1.0))


def _feat1_body(x_ref, d_ref, w_ref, out_ref):
  x = x_ref[...]
  for r in range(R):
    sn = _srcnorm(d_ref, r)
    out_ref[r] = jnp.dot(x * sn[:, None], w_ref[r],
                         preferred_element_type=jnp.float32)


def _feat1_call(x, deg2, W1):
  return pl.pallas_call(
      _feat1_body,
      grid=(NB,),
      in_specs=[
          pl.BlockSpec((MXU_BLK, D), lambda i: (i, 0)),
          pl.BlockSpec((NC * 2 * R, MXU_BLK), lambda i: (0, i)),
          pl.BlockSpec((R, D, D), lambda i: (0, 0, 0)),
      ],
      out_specs=pl.BlockSpec((R, MXU_BLK, D), lambda i: (0, i, 0)),
      out_shape=jax.ShapeDtypeStruct((R, N, D), jnp.float32),
  )(x, deg2, W1)


def _mid_body(agg_ref, d_ref, b1_ref, w_ref, out_ref):
  bsum = jnp.sum(b1_ref[...], axis=0)
  h = jnp.broadcast_to(bsum[None, :], (MXU_BLK, D))
  for r in range(R):
    h = h + agg_ref[r] * _dstnorm(d_ref, r)[:, None]
  h = jnp.maximum(h, 0.0)
  for r in range(R):
    out_ref[r] = jnp.dot(h * _srcnorm(d_ref, r)[:, None], w_ref[r],
                         preferred_element_type=jnp.float32)


def _mid_call(agg1, deg2, b1, W2):
  return pl.pallas_call(
      _mid_body,
      grid=(NB,),
      in_specs=[
          pl.BlockSpec((R, MXU_BLK, D), lambda i: (0, i, 0)),
          pl.BlockSpec((NC * 2 * R, MXU_BLK), lambda i: (0, i)),
          pl.BlockSpec((R, D), lambda i: (0, 0)),
          pl.BlockSpec((R, D, D), lambda i: (0, 0, 0)),
      ],
      out_specs=pl.BlockSpec((R, MXU_BLK, D), lambda i: (0, i, 0)),
      out_shape=jax.ShapeDtypeStruct((R, N, D), jnp.float32),
  )(agg1, deg2, b1, W2)


def _fin_body(agg_ref, d_ref, b2_ref, out_ref):
  bsum = jnp.sum(b2_ref[...], axis=0)
  o = jnp.broadcast_to(bsum[None, :], (MXU_BLK, D))
  for r in range(R):
    o = o + agg_ref[r] * _dstnorm(d_ref, r)[:, None]
  out_ref[...] = o


def _fin_call(agg2, deg2, b2):
  return pl.pallas_call(
      _fin_body,
      grid=(NB,),
      in_specs=[
          pl.BlockSpec((R, MXU_BLK, D), lambda i: (0, i, 0)),
          pl.BlockSpec((NC * 2 * R, MXU_BLK), lambda i: (0, i)),
          pl.BlockSpec((R, D), lambda i: (0, 0)),
      ],
      out_specs=pl.BlockSpec((MXU_BLK, D), lambda i: (i, 0)),
      out_shape=jax.ShapeDtypeStruct((N, D), jnp.float32),
  )(agg2, deg2, b2)


@jax.jit
def kernel(x, edge_index, W1, b1, W2, b2):
  edge_flat = edge_index.reshape(R * 2 * E)
  (degp,) = _hist_call(edge_flat)
  bsrc, bdst, counts = _bucket_call(edge_flat)
  deg2 = degp.reshape(NC * 2 * R, N_HPAD)

  feat1 = _feat1_call(x, deg2, W1).reshape(R * N, D)
  (agg1,) = _agg_call(bsrc, bdst, counts, feat1)
  feat2 = _mid_call(agg1, deg2, b1, W2).reshape(R * N, D)
  (agg2,) = _agg_call(bsrc, bdst, counts, feat2)
  return _fin_call(agg2, deg2, b2)
